# baseline (speedup 1.0000x reference)
import functools
import math

import jax
import jax.numpy as jnp
from jax import lax
from jax.experimental import pallas as pl
from jax.experimental.pallas import tpu as pltpu
from jax.experimental.pallas import tpu_sc as plsc

N = 10000
E = 320000
HID = 128
L = 4
FPAD = 16
CHUNK = 128
NCORES = 2
NSUB = 16
NW = NCORES * NSUB
NCHUNKS = E // CHUNK
SUBR = 624
ZR = 208
NPAD = 10240
SUBR1 = NPAD // NSUB
NCH2 = 2560
E2 = NCH2 * CHUNK
TCH = NCH2 // NW
GRP = 8
ZRP = 64

_mesh = functools.partial(
    plsc.VectorSubcoreMesh, core_axis_name="c", subcore_axis_name="s",
    num_cores=NCORES, num_subcores=NSUB)


def _iota16():
  return lax.iota(jnp.int32, 16)


def _zeros16f():
  return _iota16().astype(jnp.float32) * 0.0


def _lane_bcast(vec, lane):
  idx = _iota16() * 0 + lane
  return vec.at[idx].get(mode="promise_in_bounds")


def _lane_sum_all(vec):
  for sh in (1, 2, 4, 8):
    perm = lax.bitwise_xor(_iota16(), sh)
    vec = vec + vec.at[perm].get(mode="promise_in_bounds")
  return vec


def _zero_acc(acc, zbuf, s):
  base = s * SUBR
  for t in range(SUBR // ZR):
    pltpu.sync_copy(zbuf, acc.at[pl.ds(base + t * ZR, ZR)])

  @pl.when(s == NSUB - 1)
  def _():
    pltpu.sync_copy(zbuf.at[pl.ds(0, 16)], acc.at[pl.ds(N - 16, 16)])


def _flush_acc(acc, out, c, s):
  base = s * SUBR
  pltpu.sync_copy(acc.at[pl.ds(base, SUBR)], out.at[c, pl.ds(base, SUBR)])

  @pl.when(s == NSUB - 1)
  def _():
    pltpu.sync_copy(acc.at[pl.ds(N - 16, 16)], out.at[c, pl.ds(N - 16, 16)])


def _zero_acc1(acc, zbuf, s):
  pltpu.sync_copy(zbuf, acc.at[pl.ds(s * SUBR1, SUBR1)])


def _zero_acc_p(acc, zbuf, s):
  for t in range(SUBR1 // ZRP):
    pltpu.sync_copy(zbuf, acc.at[pl.ds(s * SUBR1 + t * ZRP, ZRP)])


def _flush_acc_p(acc, out, c, s):
  base = s * SUBR1
  pltpu.sync_copy(acc.at[pl.ds(base, SUBR1)], out.at[c, pl.ds(base, SUBR1)])


def _flush_acc1(acc, out0, out1, c, s):
  @pl.when((s == 0) & (c == 0))
  def _():
    pltpu.sync_copy(acc, out0)

  @pl.when((s == 0) & (c == 1))
  def _():
    pltpu.sync_copy(acc, out1)


def _zero_fill(ref, nrows, width):
  z16 = _zeros16f()

  def row(r, _):
    for j in range(width // 16):
      ref[r, pl.ds(16 * j, 16)] = z16
    return 0

  lax.fori_loop(0, nrows, row, 0)


def _zero_fill1(ref, n):
  z16 = _zeros16f()

  def row(r, _):
    ref[pl.ds(r * 16, 16)] = z16
    return 0

  lax.fori_loop(0, n // 16, row, 0)


@functools.lru_cache(None)
def _seg_accum(width):
  @functools.partial(
      pl.kernel,
      out_type=jax.ShapeDtypeStruct((NCORES, NPAD, width), jnp.float32),
      mesh=_mesh(),
      scratch_types=[
          pltpu.VMEM((2 * GRP, CHUNK), jnp.int32),
          pltpu.VMEM((GRP, CHUNK), jnp.float32),
          pltpu.VMEM((CHUNK, width), jnp.float32),
          pltpu.VMEM((ZRP, width), jnp.float32),
          pltpu.VMEM_SHARED((NPAD, width), jnp.float32),
          pltpu.SemaphoreType.DMA,
      ],
  )
  def seg(table, ipk, wpk, out, ipg, iwg, rows, zbuf, acc, sem):
    c = lax.axis_index("c")
    s = lax.axis_index("s")
    wid = s * NCORES + c

    _zero_fill(zbuf, ZRP, width)
    _zero_acc_p(acc, zbuf, s)
    plsc.subcore_barrier()

    def group(g, _):
      pltpu.sync_copy(ipk.at[wid, pl.ds(g * 2 * GRP, 2 * GRP)], ipg)
      pltpu.sync_copy(wpk.at[wid, pl.ds(g * GRP, GRP)], iwg)
      for k in range(GRP):
        pltpu.async_copy(table.at[ipg.at[2 * k]], rows, sem).wait()

        def grp16(g16, _):
          w_v = iwg[k, pl.ds(g16 * 16, 16)]
          for i in range(16):
            e = g16 * 16 + i
            wb = _lane_bcast(w_v, i)
            for j in range(width // 16):
              sl = pl.ds(16 * j, 16)
              rows[e, sl] = rows[e, sl] * wb
          return 0

        lax.fori_loop(0, CHUNK // 16, grp16, 0)
        pltpu.sync_copy(rows, acc.at[ipg.at[2 * k + 1]], add=True)
      return 0

    lax.fori_loop(0, TCH // GRP, group, 0)
    plsc.subcore_barrier()
    _flush_acc_p(acc, out, c, s)

  return seg


@functools.lru_cache(None)
def _tconv1a_sc():
  @functools.partial(
      pl.kernel,
      out_type=(jax.ShapeDtypeStruct((NW, TCH, CHUNK), jnp.float32),
                jax.ShapeDtypeStruct((4 * NPAD,), jnp.float32),
                jax.ShapeDtypeStruct((4 * NPAD,), jnp.float32)),
      mesh=_mesh(),
      scratch_types=[
          pltpu.VMEM((2 * GRP, CHUNK), jnp.int32),
          pltpu.VMEM((2 * GRP, CHUNK), jnp.float32),
          pltpu.VMEM((CHUNK, HID), jnp.float32),
          pltpu.VMEM((CHUNK, HID), jnp.float32),
          pltpu.VMEM((1, CHUNK), jnp.float32),
          pltpu.VMEM((1, CHUNK), jnp.float32),
          pltpu.VMEM((GRP, CHUNK), jnp.float32),
          pltpu.VMEM((1, CHUNK), jnp.float32),
          pltpu.VMEM((1, CHUNK), jnp.float32),
          pltpu.VMEM((3, CHUNK), jnp.int32),
          pltpu.VMEM((SUBR1,), jnp.float32),
          pltpu.VMEM_SHARED((4 * NPAD,), jnp.float32),
          pltpu.SemaphoreType.DMA,
          pltpu.SemaphoreType.DMA,
          pltpu.SemaphoreType.DMA,
          pltpu.SemaphoreType.DMA,
      ],
  )
  def tconv1a(qh, kh, qwkh, qwk1h, ipk, eapk, outE, outS0, outS1,
              ipg, eag, krows, qrows, qwkv, qwk1v, ebuf, pbuf, rbuf, idx4,
              zbufd, accS, sem0, sem1, sem2, sem3):
    c = lax.axis_index("c")
    s = lax.axis_index("s")
    wid = s * NCORES + c

    _zero_fill1(zbufd, SUBR1)
    for t in range(4):
      pltpu.sync_copy(zbufd, accS.at[pl.ds((4 * s + t) * SUBR1, SUBR1)])
    plsc.subcore_barrier()

    def group(g, _):
      pltpu.sync_copy(ipk.at[wid, pl.ds(g * 2 * GRP, 2 * GRP)], ipg)
      pltpu.sync_copy(eapk.at[wid, pl.ds(g * 2 * GRP, 2 * GRP)], eag)
      for k in range(GRP):
        cp0 = pltpu.async_copy(kh.at[ipg.at[2 * k]], krows, sem0)
        cp1 = pltpu.async_copy(qh.at[ipg.at[2 * k + 1]], qrows, sem1)
        cp2 = pltpu.async_copy(qwkh.at[ipg.at[2 * k + 1]], qwkv.at[0], sem2)
        cp3 = pltpu.async_copy(qwk1h.at[ipg.at[2 * k + 1]], qwk1v.at[0],
                               sem3)
        cp0.wait()
        cp1.wait()
        cp2.wait()
        cp3.wait()

        def edge_grp(g16, _):
          gsl = pl.ds(g16 * 16, 16)
          a0v = eag[2 * k, gsl]
          a1v = eag[2 * k + 1, gsl]
          add16 = a0v * qwkv[0, gsl] + a1v * qwk1v[0, gsl]
          dv = ipg[2 * k + 1, gsl]
          iot = _iota16()
          pk = _zeros16f()
          for i in range(16):
            e = g16 * 16 + i
            acc = _zeros16f()
            for j in range(8):
              slj = pl.ds(16 * j, 16)
              acc = acc + qrows[e, slj] * krows[e, slj]
            ev = jnp.exp(_lane_sum_all(acc) + _lane_bcast(add16, i))
            pk = jnp.where(iot == i, ev, pk)
          ebuf[k, gsl] = pk
          pbuf[0, gsl] = pk * a0v
          rbuf[0, gsl] = pk * a1v
          d4 = dv * 4
          idx4[0, gsl] = d4
          idx4[1, gsl] = d4 + 1
          idx4[2, gsl] = d4 + 2
          return 0

        lax.fori_loop(0, CHUNK // 16, edge_grp, 0)
        pltpu.sync_copy(ebuf.at[k], accS.at[idx4.at[0]], add=True)
        pltpu.sync_copy(pbuf.at[0], accS.at[idx4.at[1]], add=True)
        pltpu.sync_copy(rbuf.at[0], accS.at[idx4.at[2]], add=True)
      pltpu.sync_copy(ebuf, outE.at[wid, pl.ds(g * GRP, GRP)])
      return 0

    lax.fori_loop(0, TCH // GRP, group, 0)
    plsc.subcore_barrier()
    _flush_acc1(accS, outS0, outS1, c, s)

  return tconv1a


@functools.lru_cache(None)
def _tconv2_sc():
  @functools.partial(
      pl.kernel,
      out_type=(jax.ShapeDtypeStruct((2 * NPAD,), jnp.float32),
                jax.ShapeDtypeStruct((2 * NPAD,), jnp.float32)),
      mesh=_mesh(),
      scratch_types=[
          pltpu.VMEM((1, CHUNK), jnp.int32),
          pltpu.VMEM((1, CHUNK), jnp.int32),
          pltpu.VMEM((1, CHUNK), jnp.int32),
          pltpu.VMEM((1, CHUNK), jnp.int32),
          pltpu.VMEM((1, CHUNK), jnp.float32),
          pltpu.VMEM((1, CHUNK), jnp.float32),
          pltpu.VMEM((1, CHUNK), jnp.float32),
          pltpu.VMEM((1, CHUNK), jnp.float32),
          pltpu.VMEM((1, CHUNK), jnp.float32),
          pltpu.VMEM((1, CHUNK), jnp.float32),
          pltpu.VMEM((1, CHUNK), jnp.float32),
          pltpu.VMEM((1, 16), jnp.float32),
          pltpu.VMEM((2 * SUBR1,), jnp.float32),
          pltpu.VMEM_SHARED((2 * NPAD,), jnp.float32),
          pltpu.SemaphoreType.DMA,
          pltpu.SemaphoreType.DMA,
          pltpu.SemaphoreType.DMA,
      ],
  )
  def tconv2(qh, kh, vh, srch, dsth, ea0h, ea1h, weh, out0, out1,
             sidx, didx, d2a, d2b, ea0, ea1, kv, vv, qv, ebuf, evbuf,
             wv, zbufd, acc, sem0, sem1, sem2):
    c = lax.axis_index("c")
    s = lax.axis_index("s")
    wid = s * NCORES + c

    pltpu.sync_copy(weh, wv)
    _zero_fill1(zbufd, 2 * SUBR1)
    pltpu.sync_copy(zbufd, acc.at[pl.ds(s * 2 * SUBR1, 2 * SUBR1)])
    plsc.subcore_barrier()

    wrow = wv[0, pl.ds(0, 16)]
    wk0 = _lane_bcast(wrow, 0)
    wk1 = _lane_bcast(wrow, 1)
    wv0 = _lane_bcast(wrow, 2)
    wv1 = _lane_bcast(wrow, 3)

    nch = (NCHUNKS // NW) + jnp.where(wid < (NCHUNKS % NW), 1, 0)

    def chunk(kk, _):
      base = (wid + kk * NW) * CHUNK
      pltpu.sync_copy(srch.at[pl.ds(base, CHUNK)], sidx.at[0])
      pltpu.sync_copy(dsth.at[pl.ds(base, CHUNK)], didx.at[0])
      pltpu.sync_copy(ea0h.at[pl.ds(base, CHUNK)], ea0.at[0])
      pltpu.sync_copy(ea1h.at[pl.ds(base, CHUNK)], ea1.at[0])
      cp0 = pltpu.async_copy(kh.at[sidx.at[0]], kv.at[0], sem0)
      cp1 = pltpu.async_copy(vh.at[sidx.at[0]], vv.at[0], sem1)
      cp2 = pltpu.async_copy(qh.at[didx.at[0]], qv.at[0], sem2)
      cp0.wait()
      cp1.wait()
      cp2.wait()

      def grp(g, _):
        sl = pl.ds(g * 16, 16)
        a0 = ea0[0, sl]
        a1 = ea1[0, sl]
        dv = didx[0, sl]
        ke = kv[0, sl] + a0 * wk0 + a1 * wk1
        ve = vv[0, sl] + a0 * wv0 + a1 * wv1
        e = jnp.exp(qv[0, sl] * ke)
        ebuf[0, sl] = e
        evbuf[0, sl] = e * ve
        d2a[0, sl] = dv * 2
        d2b[0, sl] = dv * 2 + 1
        return 0

      lax.fori_loop(0, CHUNK // 16, grp, 0)
      pltpu.sync_copy(ebuf.at[0], acc.at[d2a.at[0]], add=True)
      pltpu.sync_copy(evbuf.at[0], acc.at[d2b.at[0]], add=True)
      return 0

    lax.fori_loop(0, nch, chunk, 0)
    plsc.subcore_barrier()
    _flush_acc1(acc, out0, out1, c, s)

  return tconv2


def _ln_rows(x, g, b):
  mu = jnp.mean(x, axis=-1, keepdims=True)
  xc = x - mu
  var = jnp.mean(xc * xc, axis=-1, keepdims=True)
  return xc / jnp.sqrt(var + 1e-5) * g + b


_RBLK = 2000
_GRID = N // _RBLK


def _row_spec(width):
  return pl.BlockSpec((_RBLK, width), lambda i: (i, 0))


def _pair_spec(width):
  return pl.BlockSpec((NCORES, _RBLK, width), lambda i: (0, i, 0))


def _full_spec(*shape):
  nd = len(shape)
  return pl.BlockSpec(shape, lambda i, _n=nd: (0,) * _n)


def _lstm_mid_tc(win):
  def body(inp, aggx, h, aggh, cc, wx, wh, bb, lg, lb, hn, cn):
    zx = inp[...] + aggx[0] + aggx[1]
    zh = h[...] + aggh[0] + aggh[1]
    gates = (jnp.dot(zx, wx[...], preferred_element_type=jnp.float32)
             + jnp.dot(zh, wh[...], preferred_element_type=jnp.float32)
             + bb[...])
    ii = jax.nn.sigmoid(gates[:, :HID])
    ff = jax.nn.sigmoid(gates[:, HID:2 * HID])
    gg = jnp.tanh(gates[:, 2 * HID:3 * HID])
    oo = jax.nn.sigmoid(gates[:, 3 * HID:])
    c_new = ff * cc[...] + ii * gg
    h_new = oo * jnp.tanh(c_new)
    hn[...] = _ln_rows(h_new, lg[0], lb[0])
    cn[...] = _ln_rows(c_new, lg[1], lb[1])

  return pl.pallas_call(
      body,
      grid=(_GRID,),
      in_specs=[
          _row_spec(win), _pair_spec(win), _row_spec(HID), _pair_spec(HID),
          _row_spec(HID), _full_spec(win, 4 * HID), _full_spec(HID, 4 * HID),
          _full_spec(1, 4 * HID), _full_spec(3, HID), _full_spec(3, HID),
      ],
      out_specs=[_row_spec(HID), _row_spec(HID)],
      out_shape=[jax.ShapeDtypeStruct((N, HID), jnp.float32)] * 2,
  )


def _lstm_last_tc():
  isd = 1.0 / math.sqrt(float(HID))

  def body(inp, aggx, h, aggh, cc, wx, wh, bb, lg, lb, skp, w1a, w1b, b1r,
           hn, cn, q, k, v, root):
    zx = inp[...] + aggx[0] + aggx[1]
    zh = h[...] + aggh[0] + aggh[1]
    gates = (jnp.dot(zx, wx[...], preferred_element_type=jnp.float32)
             + jnp.dot(zh, wh[...], preferred_element_type=jnp.float32)
             + bb[...])
    ii = jax.nn.sigmoid(gates[:, :HID])
    ff = jax.nn.sigmoid(gates[:, HID:2 * HID])
    gg = jnp.tanh(gates[:, 2 * HID:3 * HID])
    oo = jax.nn.sigmoid(gates[:, 3 * HID:])
    c_new = ff * cc[...] + ii * gg
    h_new = oo * jnp.tanh(c_new)
    hn[...] = _ln_rows(h_new, lg[0], lb[0])
    cn[...] = _ln_rows(c_new, lg[1], lb[1])
    out1 = jax.nn.relu(_ln_rows(h_new, lg[2], lb[2]))
    sk = skp[...]

    def proj(idx):
      return (jnp.dot(out1, w1a[idx], preferred_element_type=jnp.float32)
              + jnp.dot(sk, w1b[idx], preferred_element_type=jnp.float32))

    q[...] = proj(0) * isd
    k[...] = proj(1)
    v[...] = proj(2)
    root[...] = proj(3) + b1r[...]

  return pl.pallas_call(
      body,
      grid=(_GRID,),
      in_specs=[
          _row_spec(HID), _pair_spec(HID), _row_spec(HID), _pair_spec(HID),
          _row_spec(HID), _full_spec(HID, 4 * HID), _full_spec(HID, 4 * HID),
          _full_spec(1, 4 * HID), _full_spec(3, HID), _full_spec(3, HID),
          _row_spec(FPAD), _full_spec(4, HID, HID), _full_spec(4, FPAD, HID),
          _full_spec(1, HID),
      ],
      out_specs=[_row_spec(HID)] * 6,
      out_shape=[jax.ShapeDtypeStruct((N, HID), jnp.float32)] * 6,
  )


def _qwe_tc():
  def body(q, wef, qwk, qwk1):
    qwk[...] = jnp.sum(q[...] * wef[0][None, :], axis=1)
    qwk1[...] = jnp.sum(q[...] * wef[1][None, :], axis=1)

  return pl.pallas_call(
      body,
      grid=(1,),
      in_specs=[_full_spec(N, HID), _full_spec(4, HID)],
      out_specs=[_full_spec(N)] * 2,
      out_shape=[jax.ShapeDtypeStruct((N,), jnp.float32)] * 2,
  )


def _head2_tc():
  def body(evp, dnp, pp, rr, root1, wef, w2r, b2r, q2, k2, v2, root2):
    den = dnp[0] + dnp[1]
    psum = pp[0] + pp[1]
    rsum = rr[0] + rr[1]
    ev = (evp[0] + evp[1] + psum[:, None] * wef[2][None, :]
          + rsum[:, None] * wef[3][None, :])
    t1 = jax.nn.relu(ev / (den[:, None] + 1e-16) + root1[...])
    q2[...] = jnp.sum(t1 * w2r[0][None, :], axis=1)
    k2[...] = jnp.sum(t1 * w2r[1][None, :], axis=1)
    v2[...] = jnp.sum(t1 * w2r[2][None, :], axis=1)
    root2[...] = jnp.sum(t1 * w2r[3][None, :], axis=1) + b2r[0, 0]

  return pl.pallas_call(
      body,
      grid=(1,),
      in_specs=[
          _full_spec(NCORES, N, HID), _full_spec(NCORES, N),
          _full_spec(NCORES, N), _full_spec(NCORES, N),
          _full_spec(N, HID), _full_spec(4, HID), _full_spec(4, HID),
          _full_spec(1, 1),
      ],
      out_specs=[_full_spec(N)] * 4,
      out_shape=[jax.ShapeDtypeStruct((N,), jnp.float32)] * 4,
  )


def _final_tc():
  def body(dnp, evp, root2, out):
    den = dnp[0] + dnp[1]
    ev = evp[0] + evp[1]
    out[...] = jax.nn.sigmoid(ev / (den + 1e-16) + root2[...])

  return pl.pallas_call(
      body,
      grid=(1,),
      in_specs=[_full_spec(NCORES, N), _full_spec(NCORES, N), _full_spec(N)],
      out_specs=_full_spec(N),
      out_shape=jax.ShapeDtypeStruct((N,), jnp.float32),
  )


def kernel(X, edge_weight, skip, H, C, Wx0, Wxs, Wh, b_lstm, ln_g, ln_b,
           W1, We1, b1, W2, We2, b2, edge_index):
  src = edge_index[0]
  dst = edge_index[1]
  w = edge_weight[:, 0]
  ea1 = edge_weight[:, 1]

  pad = E2 - E
  srcp = jnp.pad(src, (0, pad))
  dstp = jnp.concatenate(
      [dst, N + (jnp.arange(pad, dtype=jnp.int32) % (NPAD - N))])
  wp = jnp.pad(w, (0, pad))
  ea1p = jnp.pad(ea1, (0, pad))
  srct = srcp.reshape(TCH, NW, CHUNK).transpose(1, 0, 2)
  dstt = dstp.reshape(TCH, NW, CHUNK).transpose(1, 0, 2)
  pack = jnp.stack([srct, dstt], axis=2).reshape(NW, 2 * TCH, CHUNK)
  wpack = wp.reshape(TCH, NW, CHUNK).transpose(1, 0, 2)
  ea1t = ea1p.reshape(TCH, NW, CHUNK).transpose(1, 0, 2)

  x = X[0]
  xpad = jnp.pad(x, ((0, 0), (0, HID - x.shape[1])))
  wx0p = jnp.pad(Wx0, ((0, HID - Wx0.shape[0]), (0, 0)))
  skp = jnp.pad(skip, ((0, 0), (0, FPAD - skip.shape[1])))
  w1a = W1[:, :HID, :]
  w1b = jnp.pad(W1[:, HID:, :], ((0, 0), (0, FPAD - 2), (0, 0)))
  we1f = We1.reshape(4, HID)
  w2r = W2.reshape(4, HID)
  we2f = jnp.pad(We2.reshape(4), (0, 12)).reshape(1, 16)
  b2r = b2.reshape(1, 1)
  b1r = b1.reshape(1, HID)

  seg128 = _seg_accum(HID)

  hs, cs = [], []
  inp, aggi = xpad, seg128(xpad, pack, wpack)
  q = k = v = root1 = None
  for l in range(L):
    aggh = seg128(H[l], pack, wpack)
    wx = wx0p if l == 0 else Wxs[l - 1]
    blk = b_lstm[l].reshape(1, 4 * HID)
    if l < L - 1:
      hn, cn = _lstm_mid_tc(HID)(inp, aggi, H[l], aggh, C[l], wx, Wh[l],
                                 blk, ln_g, ln_b)
      aggi, inp = seg128(hn, pack, wpack), hn
    else:
      hn, cn, q, k, v, root1 = _lstm_last_tc()(
          inp, aggi, H[l], aggh, C[l], wx, Wh[l], blk, ln_g, ln_b,
          skp, w1a, w1b, b1r)
    hs.append(hn)
    cs.append(cn)

  qwk_n, qwk1_n = _qwe_tc()(q, we1f)
  qpad = jnp.pad(q, ((0, NPAD - N), (0, 0)))
  qwk = jnp.pad(qwk_n, (0, NPAD - N))
  qwk1 = jnp.pad(qwk1_n, (0, NPAD - N))
  eapk = jnp.stack([wpack, ea1t], axis=2).reshape(NW, 2 * TCH, CHUNK)
  epk, s0, s1 = _tconv1a_sc()(qpad, k, qwk, qwk1, pack, eapk)
  evp = seg128(v, pack, epk)
  dnp = jnp.stack([s0[0::4][:N], s1[0::4][:N]])
  pp = jnp.stack([s0[1::4][:N], s1[1::4][:N]])
  rr = jnp.stack([s0[2::4][:N], s1[2::4][:N]])
  q2, k2, v2, root2 = _head2_tc()(evp, dnp, pp, rr, root1, we1f, w2r, b2r)
  de0, de1 = _tconv2_sc()(q2, k2, v2, src, dst, w, ea1, we2f)
  out = _final_tc()(jnp.stack([de0[0::2][:N], de1[0::2][:N]]),
                    jnp.stack([de0[1::2][:N], de1[1::2][:N]]), root2)
  return (out.reshape(N, 1), jnp.stack(hs), jnp.stack(cs))


# speedup vs baseline: 1.4619x; 1.4619x over previous
import functools
import math

import jax
import jax.numpy as jnp
from jax import lax
from jax.experimental import pallas as pl
from jax.experimental.pallas import tpu as pltpu
from jax.experimental.pallas import tpu_sc as plsc

N = 10000
E = 320000
HID = 128
L = 4
FPAD = 16
CHUNK = 128
NCORES = 2
NSUB = 16
NW = NCORES * NSUB
NCHUNKS = E // CHUNK
SUBR = 624
ZR = 208
NPAD = 10240
SUBR1 = NPAD // NSUB
NCH2 = 2560
E2 = NCH2 * CHUNK
TCH = NCH2 // NW
GRP = 8
ZRP = 64

_mesh = functools.partial(
    plsc.VectorSubcoreMesh, core_axis_name="c", subcore_axis_name="s",
    num_cores=NCORES, num_subcores=NSUB)


def _iota16():
  return lax.iota(jnp.int32, 16)


def _zeros16f():
  return _iota16().astype(jnp.float32) * 0.0


def _lane_bcast(vec, lane):
  idx = _iota16() * 0 + lane
  return vec.at[idx].get(mode="promise_in_bounds")


def _lane_sum_all(vec):
  for sh in (1, 2, 4, 8):
    perm = lax.bitwise_xor(_iota16(), sh)
    vec = vec + vec.at[perm].get(mode="promise_in_bounds")
  return vec


def _zero_acc(acc, zbuf, s):
  base = s * SUBR
  for t in range(SUBR // ZR):
    pltpu.sync_copy(zbuf, acc.at[pl.ds(base + t * ZR, ZR)])

  @pl.when(s == NSUB - 1)
  def _():
    pltpu.sync_copy(zbuf.at[pl.ds(0, 16)], acc.at[pl.ds(N - 16, 16)])


def _flush_acc(acc, out, c, s):
  base = s * SUBR
  pltpu.sync_copy(acc.at[pl.ds(base, SUBR)], out.at[c, pl.ds(base, SUBR)])

  @pl.when(s == NSUB - 1)
  def _():
    pltpu.sync_copy(acc.at[pl.ds(N - 16, 16)], out.at[c, pl.ds(N - 16, 16)])


def _zero_acc1(acc, zbuf, s):
  pltpu.sync_copy(zbuf, acc.at[pl.ds(s * SUBR1, SUBR1)])


def _zero_acc_p(acc, zbuf, s):
  for t in range(SUBR1 // ZRP):
    pltpu.sync_copy(zbuf, acc.at[pl.ds(s * SUBR1 + t * ZRP, ZRP)])


def _flush_acc_p(acc, out, c, s):
  base = s * SUBR1
  pltpu.sync_copy(acc.at[pl.ds(base, SUBR1)], out.at[c, pl.ds(base, SUBR1)])


def _flush_acc1(acc, out0, out1, c, s):
  @pl.when((s == 0) & (c == 0))
  def _():
    pltpu.sync_copy(acc, out0)

  @pl.when((s == 0) & (c == 1))
  def _():
    pltpu.sync_copy(acc, out1)


def _zero_fill(ref, nrows, width):
  z16 = _zeros16f()

  def row(r, _):
    for j in range(width // 16):
      ref[r, pl.ds(16 * j, 16)] = z16
    return 0

  lax.fori_loop(0, nrows, row, 0)


def _zero_fill1(ref, n):
  z16 = _zeros16f()

  def row(r, _):
    ref[pl.ds(r * 16, 16)] = z16
    return 0

  lax.fori_loop(0, n // 16, row, 0)


@functools.lru_cache(None)
def _seg_accum(width):
  @functools.partial(
      pl.kernel,
      out_type=jax.ShapeDtypeStruct((NCORES, N, width), jnp.float32),
      mesh=_mesh(),
      scratch_types=[
          pltpu.VMEM((1, CHUNK), jnp.int32),
          pltpu.VMEM((1, CHUNK), jnp.int32),
          pltpu.VMEM((1, CHUNK), jnp.float32),
          pltpu.VMEM((CHUNK, width), jnp.float32),
          pltpu.VMEM((ZR, width), jnp.float32),
          pltpu.VMEM_SHARED((N, width), jnp.float32),
          pltpu.SemaphoreType.DMA,
      ],
  )
  def seg(table, srch, dsth, wh, out, sidx, didx, wv, rows, zbuf, acc, sem):
    c = lax.axis_index("c")
    s = lax.axis_index("s")
    wid = s * NCORES + c

    _zero_fill(zbuf, ZR, width)
    _zero_acc(acc, zbuf, s)
    plsc.subcore_barrier()

    nch = (NCHUNKS // NW) + jnp.where(wid < (NCHUNKS % NW), 1, 0)

    def chunk(kk, _):
      base = (wid + kk * NW) * CHUNK
      pltpu.sync_copy(srch.at[pl.ds(base, CHUNK)], sidx.at[0])
      pltpu.sync_copy(dsth.at[pl.ds(base, CHUNK)], didx.at[0])
      pltpu.sync_copy(wh.at[pl.ds(base, CHUNK)], wv.at[0])
      pltpu.async_copy(table.at[sidx.at[0]], rows, sem).wait()

      def scale(g, _):
        w_v = wv[0, pl.ds(g * 16, 16)]
        for i in range(16):
          e = g * 16 + i
          wb = _lane_bcast(w_v, i)
          for j in range(width // 16):
            sl = pl.ds(16 * j, 16)
            rows[e, sl] = rows[e, sl] * wb
        return 0

      lax.fori_loop(0, CHUNK // 16, scale, 0)
      pltpu.sync_copy(rows, acc.at[didx.at[0]], add=True)
      return 0

    lax.fori_loop(0, nch, chunk, 0)
    plsc.subcore_barrier()
    _flush_acc(acc, out, c, s)

  return seg


@functools.lru_cache(None)
def _tconv1a_sc():
  @functools.partial(
      pl.kernel,
      out_type=(jax.ShapeDtypeStruct((E,), jnp.float32),
                jax.ShapeDtypeStruct((NPAD,), jnp.float32),
                jax.ShapeDtypeStruct((NPAD,), jnp.float32)),
      mesh=_mesh(),
      scratch_types=[
          pltpu.VMEM((1, CHUNK), jnp.int32),
          pltpu.VMEM((1, CHUNK), jnp.int32),
          pltpu.VMEM((1, CHUNK), jnp.float32),
          pltpu.VMEM((1, CHUNK), jnp.float32),
          pltpu.VMEM((CHUNK, HID), jnp.float32),
          pltpu.VMEM((CHUNK, HID), jnp.float32),
          pltpu.VMEM((1, CHUNK), jnp.float32),
          pltpu.VMEM((4, HID), jnp.float32),
          pltpu.VMEM((SUBR1,), jnp.float32),
          pltpu.VMEM_SHARED((NPAD,), jnp.float32),
          pltpu.SemaphoreType.DMA,
          pltpu.SemaphoreType.DMA,
      ],
  )
  def tconv1a(qh, kh, srch, dsth, ea0h, ea1h, weh, outE, outD0, outD1,
              sidx, didx, ea0, ea1, krows, qrows, ebuf, wev, zbufd,
              accD, sem0, sem1):
    c = lax.axis_index("c")
    s = lax.axis_index("s")
    wid = s * NCORES + c

    pltpu.sync_copy(weh, wev)
    _zero_fill1(zbufd, SUBR1)
    _zero_acc1(accD, zbufd, s)
    plsc.subcore_barrier()

    wek = [wev[0, pl.ds(16 * j, 16)] for j in range(8)]
    wek1 = [wev[1, pl.ds(16 * j, 16)] for j in range(8)]

    nch = (NCHUNKS // NW) + jnp.where(wid < (NCHUNKS % NW), 1, 0)

    def chunk(kk, _):
      base = (wid + kk * NW) * CHUNK
      pltpu.sync_copy(srch.at[pl.ds(base, CHUNK)], sidx.at[0])
      pltpu.sync_copy(dsth.at[pl.ds(base, CHUNK)], didx.at[0])
      pltpu.sync_copy(ea0h.at[pl.ds(base, CHUNK)], ea0.at[0])
      pltpu.sync_copy(ea1h.at[pl.ds(base, CHUNK)], ea1.at[0])
      cp0 = pltpu.async_copy(kh.at[sidx.at[0]], krows, sem0)
      cp1 = pltpu.async_copy(qh.at[didx.at[0]], qrows, sem1)
      cp0.wait()
      cp1.wait()

      def edge_grp(g, _):
        gsl = pl.ds(g * 16, 16)
        a0v = ea0[0, gsl]
        a1v = ea1[0, gsl]
        iot = _iota16()
        pk = _zeros16f()
        for i in range(16):
          e = g * 16 + i
          a0 = _lane_bcast(a0v, i)
          a1 = _lane_bcast(a1v, i)
          acc = _zeros16f()
          for j in range(8):
            sl = pl.ds(16 * j, 16)
            ke = krows[e, sl] + a0 * wek[j] + a1 * wek1[j]
            acc = acc + qrows[e, sl] * ke
          ev = jnp.exp(_lane_sum_all(acc))
          pk = jnp.where(iot == i, ev, pk)
        ebuf[0, gsl] = pk
        return 0

      lax.fori_loop(0, CHUNK // 16, edge_grp, 0)
      pltpu.sync_copy(ebuf.at[0], outE.at[pl.ds(base, CHUNK)])
      pltpu.sync_copy(ebuf.at[0], accD.at[didx.at[0]], add=True)
      return 0

    lax.fori_loop(0, nch, chunk, 0)
    plsc.subcore_barrier()
    _flush_acc1(accD, outD0, outD1, c, s)

  return tconv1a


@functools.lru_cache(None)
def _tconv1b_sc():
  @functools.partial(
      pl.kernel,
      out_type=jax.ShapeDtypeStruct((NCORES, N, HID), jnp.float32),
      mesh=_mesh(),
      scratch_types=[
          pltpu.VMEM((1, CHUNK), jnp.int32),
          pltpu.VMEM((1, CHUNK), jnp.int32),
          pltpu.VMEM((1, CHUNK), jnp.float32),
          pltpu.VMEM((1, CHUNK), jnp.float32),
          pltpu.VMEM((1, CHUNK), jnp.float32),
          pltpu.VMEM((CHUNK, HID), jnp.float32),
          pltpu.VMEM((4, HID), jnp.float32),
          pltpu.VMEM((ZR, HID), jnp.float32),
          pltpu.VMEM_SHARED((N, HID), jnp.float32),
          pltpu.SemaphoreType.DMA,
      ],
  )
  def tconv1b(vh, eh, srch, dsth, ea0h, ea1h, weh, outV,
              sidx, didx, ea0, ea1, ebuf, vrows, wev, zbuf, accV, sem0):
    c = lax.axis_index("c")
    s = lax.axis_index("s")
    wid = s * NCORES + c

    pltpu.sync_copy(weh, wev)
    _zero_fill(zbuf, ZR, HID)
    _zero_acc(accV, zbuf, s)
    plsc.subcore_barrier()

    wevv = [wev[2, pl.ds(16 * j, 16)] for j in range(8)]
    wev1 = [wev[3, pl.ds(16 * j, 16)] for j in range(8)]

    nch = (NCHUNKS // NW) + jnp.where(wid < (NCHUNKS % NW), 1, 0)

    def chunk(kk, _):
      base = (wid + kk * NW) * CHUNK
      pltpu.sync_copy(srch.at[pl.ds(base, CHUNK)], sidx.at[0])
      pltpu.sync_copy(dsth.at[pl.ds(base, CHUNK)], didx.at[0])
      pltpu.sync_copy(ea0h.at[pl.ds(base, CHUNK)], ea0.at[0])
      pltpu.sync_copy(ea1h.at[pl.ds(base, CHUNK)], ea1.at[0])
      pltpu.sync_copy(eh.at[pl.ds(base, CHUNK)], ebuf.at[0])
      pltpu.async_copy(vh.at[sidx.at[0]], vrows, sem0).wait()

      def edge_grp(g, _):
        gsl = pl.ds(g * 16, 16)
        a0v = ea0[0, gsl]
        a1v = ea1[0, gsl]
        epk = ebuf[0, gsl]
        for i in range(16):
          e = g * 16 + i
          a0 = _lane_bcast(a0v, i)
          a1 = _lane_bcast(a1v, i)
          eb = _lane_bcast(epk, i)
          for j in range(8):
            sl = pl.ds(16 * j, 16)
            vrows[e, sl] = (vrows[e, sl] + a0 * wevv[j] + a1 * wev1[j]) * eb
        return 0

      lax.fori_loop(0, CHUNK // 16, edge_grp, 0)
      pltpu.sync_copy(vrows, accV.at[didx.at[0]], add=True)
      return 0

    lax.fori_loop(0, nch, chunk, 0)
    plsc.subcore_barrier()
    _flush_acc(accV, outV, c, s)

  return tconv1b


@functools.lru_cache(None)
def _tconv2_sc():
  @functools.partial(
      pl.kernel,
      out_type=(jax.ShapeDtypeStruct((2 * NPAD,), jnp.float32),
                jax.ShapeDtypeStruct((2 * NPAD,), jnp.float32)),
      mesh=_mesh(),
      scratch_types=[
          pltpu.VMEM((1, CHUNK), jnp.int32),
          pltpu.VMEM((1, CHUNK), jnp.int32),
          pltpu.VMEM((1, CHUNK), jnp.int32),
          pltpu.VMEM((1, CHUNK), jnp.int32),
          pltpu.VMEM((1, CHUNK), jnp.float32),
          pltpu.VMEM((1, CHUNK), jnp.float32),
          pltpu.VMEM((1, CHUNK), jnp.float32),
          pltpu.VMEM((1, CHUNK), jnp.float32),
          pltpu.VMEM((1, CHUNK), jnp.float32),
          pltpu.VMEM((1, CHUNK), jnp.float32),
          pltpu.VMEM((1, CHUNK), jnp.float32),
          pltpu.VMEM((1, 16), jnp.float32),
          pltpu.VMEM((2 * SUBR1,), jnp.float32),
          pltpu.VMEM_SHARED((2 * NPAD,), jnp.float32),
          pltpu.SemaphoreType.DMA,
          pltpu.SemaphoreType.DMA,
          pltpu.SemaphoreType.DMA,
      ],
  )
  def tconv2(qh, kh, vh, srch, dsth, ea0h, ea1h, weh, out0, out1,
             sidx, didx, d2a, d2b, ea0, ea1, kv, vv, qv, ebuf, evbuf,
             wv, zbufd, acc, sem0, sem1, sem2):
    c = lax.axis_index("c")
    s = lax.axis_index("s")
    wid = s * NCORES + c

    pltpu.sync_copy(weh, wv)
    _zero_fill1(zbufd, 2 * SUBR1)
    pltpu.sync_copy(zbufd, acc.at[pl.ds(s * 2 * SUBR1, 2 * SUBR1)])
    plsc.subcore_barrier()

    wrow = wv[0, pl.ds(0, 16)]
    wk0 = _lane_bcast(wrow, 0)
    wk1 = _lane_bcast(wrow, 1)
    wv0 = _lane_bcast(wrow, 2)
    wv1 = _lane_bcast(wrow, 3)

    nch = (NCHUNKS // NW) + jnp.where(wid < (NCHUNKS % NW), 1, 0)

    def chunk(kk, _):
      base = (wid + kk * NW) * CHUNK
      pltpu.sync_copy(srch.at[pl.ds(base, CHUNK)], sidx.at[0])
      pltpu.sync_copy(dsth.at[pl.ds(base, CHUNK)], didx.at[0])
      pltpu.sync_copy(ea0h.at[pl.ds(base, CHUNK)], ea0.at[0])
      pltpu.sync_copy(ea1h.at[pl.ds(base, CHUNK)], ea1.at[0])
      cp0 = pltpu.async_copy(kh.at[sidx.at[0]], kv.at[0], sem0)
      cp1 = pltpu.async_copy(vh.at[sidx.at[0]], vv.at[0], sem1)
      cp2 = pltpu.async_copy(qh.at[didx.at[0]], qv.at[0], sem2)
      cp0.wait()
      cp1.wait()
      cp2.wait()

      def grp(g, _):
        sl = pl.ds(g * 16, 16)
        a0 = ea0[0, sl]
        a1 = ea1[0, sl]
        dv = didx[0, sl]
        ke = kv[0, sl] + a0 * wk0 + a1 * wk1
        ve = vv[0, sl] + a0 * wv0 + a1 * wv1
        e = jnp.exp(qv[0, sl] * ke)
        ebuf[0, sl] = e
        evbuf[0, sl] = e * ve
        d2a[0, sl] = dv * 2
        d2b[0, sl] = dv * 2 + 1
        return 0

      lax.fori_loop(0, CHUNK // 16, grp, 0)
      pltpu.sync_copy(ebuf.at[0], acc.at[d2a.at[0]], add=True)
      pltpu.sync_copy(evbuf.at[0], acc.at[d2b.at[0]], add=True)
      return 0

    lax.fori_loop(0, nch, chunk, 0)
    plsc.subcore_barrier()
    _flush_acc1(acc, out0, out1, c, s)

  return tconv2


def _ln_rows(x, g, b):
  mu = jnp.mean(x, axis=-1, keepdims=True)
  xc = x - mu
  var = jnp.mean(xc * xc, axis=-1, keepdims=True)
  return xc / jnp.sqrt(var + 1e-5) * g + b


_RBLK = 2000
_GRID = N // _RBLK


def _row_spec(width):
  return pl.BlockSpec((_RBLK, width), lambda i: (i, 0))


def _pair_spec(width):
  return pl.BlockSpec((NCORES, _RBLK, width), lambda i: (0, i, 0))


def _full_spec(*shape):
  nd = len(shape)
  return pl.BlockSpec(shape, lambda i, _n=nd: (0,) * _n)


def _lstm_mid_tc(win):
  def body(inp, aggx, h, aggh, cc, wx, wh, bb, lg, lb, hn, cn):
    zx = inp[...] + aggx[0] + aggx[1]
    zh = h[...] + aggh[0] + aggh[1]
    gates = (jnp.dot(zx, wx[...], preferred_element_type=jnp.float32)
             + jnp.dot(zh, wh[...], preferred_element_type=jnp.float32)
             + bb[...])
    ii = jax.nn.sigmoid(gates[:, :HID])
    ff = jax.nn.sigmoid(gates[:, HID:2 * HID])
    gg = jnp.tanh(gates[:, 2 * HID:3 * HID])
    oo = jax.nn.sigmoid(gates[:, 3 * HID:])
    c_new = ff * cc[...] + ii * gg
    h_new = oo * jnp.tanh(c_new)
    hn[...] = _ln_rows(h_new, lg[0], lb[0])
    cn[...] = _ln_rows(c_new, lg[1], lb[1])

  return pl.pallas_call(
      body,
      grid=(_GRID,),
      in_specs=[
          _row_spec(win), _pair_spec(win), _row_spec(HID), _pair_spec(HID),
          _row_spec(HID), _full_spec(win, 4 * HID), _full_spec(HID, 4 * HID),
          _full_spec(1, 4 * HID), _full_spec(3, HID), _full_spec(3, HID),
      ],
      out_specs=[_row_spec(HID), _row_spec(HID)],
      out_shape=[jax.ShapeDtypeStruct((N, HID), jnp.float32)] * 2,
  )


def _lstm_last_tc():
  isd = 1.0 / math.sqrt(float(HID))

  def body(inp, aggx, h, aggh, cc, wx, wh, bb, lg, lb, skp, w1a, w1b, b1r,
           hn, cn, q, k, v, root):
    zx = inp[...] + aggx[0] + aggx[1]
    zh = h[...] + aggh[0] + aggh[1]
    gates = (jnp.dot(zx, wx[...], preferred_element_type=jnp.float32)
             + jnp.dot(zh, wh[...], preferred_element_type=jnp.float32)
             + bb[...])
    ii = jax.nn.sigmoid(gates[:, :HID])
    ff = jax.nn.sigmoid(gates[:, HID:2 * HID])
    gg = jnp.tanh(gates[:, 2 * HID:3 * HID])
    oo = jax.nn.sigmoid(gates[:, 3 * HID:])
    c_new = ff * cc[...] + ii * gg
    h_new = oo * jnp.tanh(c_new)
    hn[...] = _ln_rows(h_new, lg[0], lb[0])
    cn[...] = _ln_rows(c_new, lg[1], lb[1])
    out1 = jax.nn.relu(_ln_rows(h_new, lg[2], lb[2]))
    sk = skp[...]

    def proj(idx):
      return (jnp.dot(out1, w1a[idx], preferred_element_type=jnp.float32)
              + jnp.dot(sk, w1b[idx], preferred_element_type=jnp.float32))

    q[...] = proj(0) * isd
    k[...] = proj(1)
    v[...] = proj(2)
    root[...] = proj(3) + b1r[...]

  return pl.pallas_call(
      body,
      grid=(_GRID,),
      in_specs=[
          _row_spec(HID), _pair_spec(HID), _row_spec(HID), _pair_spec(HID),
          _row_spec(HID), _full_spec(HID, 4 * HID), _full_spec(HID, 4 * HID),
          _full_spec(1, 4 * HID), _full_spec(3, HID), _full_spec(3, HID),
          _row_spec(FPAD), _full_spec(4, HID, HID), _full_spec(4, FPAD, HID),
          _full_spec(1, HID),
      ],
      out_specs=[_row_spec(HID)] * 6,
      out_shape=[jax.ShapeDtypeStruct((N, HID), jnp.float32)] * 6,
  )


def _head2_tc():
  def body(evp, dnp, root1, w2r, b2r, q2, k2, v2, root2):
    den = dnp[0] + dnp[1]
    ev = evp[0] + evp[1]
    t1 = jax.nn.relu(ev / (den[:, None] + 1e-16) + root1[...])
    q2[...] = jnp.sum(t1 * w2r[0][None, :], axis=1)
    k2[...] = jnp.sum(t1 * w2r[1][None, :], axis=1)
    v2[...] = jnp.sum(t1 * w2r[2][None, :], axis=1)
    root2[...] = jnp.sum(t1 * w2r[3][None, :], axis=1) + b2r[0, 0]

  return pl.pallas_call(
      body,
      grid=(1,),
      in_specs=[
          _full_spec(NCORES, N, HID), _full_spec(NCORES, N),
          _full_spec(N, HID), _full_spec(4, HID), _full_spec(1, 1),
      ],
      out_specs=[_full_spec(N)] * 4,
      out_shape=[jax.ShapeDtypeStruct((N,), jnp.float32)] * 4,
  )


def _final_tc():
  def body(dnp, evp, root2, out):
    den = dnp[0] + dnp[1]
    ev = evp[0] + evp[1]
    out[...] = jax.nn.sigmoid(ev / (den + 1e-16) + root2[...])

  return pl.pallas_call(
      body,
      grid=(1,),
      in_specs=[_full_spec(NCORES, N), _full_spec(NCORES, N), _full_spec(N)],
      out_specs=_full_spec(N),
      out_shape=jax.ShapeDtypeStruct((N,), jnp.float32),
  )


def kernel(X, edge_weight, skip, H, C, Wx0, Wxs, Wh, b_lstm, ln_g, ln_b,
           W1, We1, b1, W2, We2, b2, edge_index):
  src = edge_index[0]
  dst = edge_index[1]
  w = edge_weight[:, 0]
  ea1 = edge_weight[:, 1]

  x = X[0]
  xpad = jnp.pad(x, ((0, 0), (0, HID - x.shape[1])))
  wx0p = jnp.pad(Wx0, ((0, HID - Wx0.shape[0]), (0, 0)))
  skp = jnp.pad(skip, ((0, 0), (0, FPAD - skip.shape[1])))
  w1a = W1[:, :HID, :]
  w1b = jnp.pad(W1[:, HID:, :], ((0, 0), (0, FPAD - 2), (0, 0)))
  we1f = We1.reshape(4, HID)
  w2r = W2.reshape(4, HID)
  we2f = jnp.pad(We2.reshape(4), (0, 12)).reshape(1, 16)
  b2r = b2.reshape(1, 1)
  b1r = b1.reshape(1, HID)

  seg128 = _seg_accum(HID)

  hs, cs = [], []
  inp, aggi = xpad, seg128(xpad, src, dst, w)
  q = k = v = root1 = None
  for l in range(L):
    aggh = seg128(H[l], src, dst, w)
    wx = wx0p if l == 0 else Wxs[l - 1]
    blk = b_lstm[l].reshape(1, 4 * HID)
    if l < L - 1:
      hn, cn = _lstm_mid_tc(HID)(inp, aggi, H[l], aggh, C[l], wx, Wh[l],
                                 blk, ln_g, ln_b)
      aggi, inp = seg128(hn, src, dst, w), hn
    else:
      hn, cn, q, k, v, root1 = _lstm_last_tc()(
          inp, aggi, H[l], aggh, C[l], wx, Wh[l], blk, ln_g, ln_b,
          skp, w1a, w1b, b1r)
    hs.append(hn)
    cs.append(cn)

  eh, dn0, dn1 = _tconv1a_sc()(q, k, src, dst, w, ea1, we1f)
  evp = _tconv1b_sc()(v, eh, src, dst, w, ea1, we1f)
  dnp = jnp.stack([dn0[:N], dn1[:N]])
  q2, k2, v2, root2 = _head2_tc()(evp, dnp, root1, w2r, b2r)
  de0, de1 = _tconv2_sc()(q2, k2, v2, src, dst, w, ea1, we2f)
  out = _final_tc()(jnp.stack([de0[0::2][:N], de1[0::2][:N]]),
                    jnp.stack([de0[1::2][:N], de1[1::2][:N]]), root2)
  return (out.reshape(N, 1), jnp.stack(hs), jnp.stack(cs))
